# baseline (speedup 1.0000x reference)
import jax
import jax.numpy as jnp
from jax.experimental import pallas as pl

_HI = jax.lax.Precision.HIGHEST


def _mm(a, b):
    return jnp.dot(a, b, precision=_HI, preferred_element_type=jnp.float32)


V = 20000
P = 32
COUT = 64
M = 15
SIGMA = 0.3
EPS = 1e-5
TILE_V = 400
TILE_R = 2000
N1 = V * P


def _build_features(vox, coors_f, npv):
    ax = coors_f[:, 3:4] * 0.2 + (0.1 - 51.2)
    ay = coors_f[:, 2:3] * 0.2 + (0.1 - 51.2)
    az = coors_f[:, 1:2] * 8.0 + (4.0 - 5.0)
    anchors = jnp.concatenate([ax, ay, az], axis=1)
    xyz_raw = vox[:, :, 0:3]
    empty = ((vox[:, :, 0:1] == 0.0) & (vox[:, :, 1:2] == 0.0)
             & (vox[:, :, 2:3] == 0.0))
    xyz = jnp.where(empty, anchors[:, None, :], xyz_raw)
    feats = vox[:, :, 3:7]
    adiff = xyz - anchors[:, None, :]
    cent = jnp.mean(xyz, axis=1, keepdims=True)
    cdiff = xyz - cent
    cexp = jnp.broadcast_to(cent, xyz.shape)
    npx = jnp.broadcast_to(npv[:, :, None], (xyz.shape[0], P, 1))
    x11 = jnp.concatenate(
        [feats, adiff[:, :, :2], cdiff[:, :, :2], cexp[:, :, :2], npx], axis=2)
    return xyz, x11, anchors


def _p1_body(vox_ref, coors_ref, npv_ref, w1_ref, b1_ref,
             anch_ref, s_ref, ss_ref):
    i = pl.program_id(0)
    coors_f = coors_ref[...].astype(jnp.float32)
    _, x11_3, anchors = _build_features(vox_ref[...], coors_f, npv_ref[...])
    anch_ref[...] = anchors
    x11 = x11_3.reshape(TILE_V * P, 11)
    u = _mm(x11, w1_ref[...]) + b1_ref[...]
    su = jnp.sum(u, axis=0, keepdims=True)
    ssq = jnp.sum(u * u, axis=0, keepdims=True)

    @pl.when(i == 0)
    def _():
        s_ref[...] = su
        ss_ref[...] = ssq

    @pl.when(i > 0)
    def _():
        s_ref[...] = s_ref[...] + su
        ss_ref[...] = ss_ref[...] + ssq


def _p2_body(vox_ref, coors_ref, npv_ref, w1_ref, b1_ref, sc1_ref, sh1_ref,
             kpx_ref, kpy_ref, kpz_ref, wkf_ref, out_ref, s_ref, ss_ref):
    i = pl.program_id(0)
    coors_f = coors_ref[...].astype(jnp.float32)
    xyz, x11_3, anchors = _build_features(vox_ref[...], coors_f, npv_ref[...])
    x11 = x11_3.reshape(TILE_V * P, 11)
    u = _mm(x11, w1_ref[...]) + b1_ref[...]
    h = jnp.maximum(u * sc1_ref[...] + sh1_ref[...], 0.0)
    rel = (xyz - anchors[:, None, :]).reshape(TILE_V * P, 3)
    dx = rel[:, 0:1] - kpx_ref[...]
    dy = rel[:, 1:2] - kpy_ref[...]
    dz = rel[:, 2:3] - kpz_ref[...]
    d2 = dx * dx + dy * dy + dz * dz
    d = jnp.sqrt(d2)
    w = jnp.maximum(1.0 - d * (1.0 / SIGMA), 0.0)
    h3 = h.reshape(TILE_V, P, COUT)
    w3 = w.reshape(TILE_V, P, M)
    aggs = [jnp.sum(w3[:, :, m:m + 1] * h3, axis=1) for m in range(M)]
    agg = jnp.concatenate(aggs, axis=1)
    out = _mm(agg, wkf_ref[...])
    out_ref[...] = out
    su = jnp.sum(out, axis=0, keepdims=True)
    ssq = jnp.sum(out * out, axis=0, keepdims=True)

    @pl.when(i == 0)
    def _():
        s_ref[...] = su
        ss_ref[...] = ssq

    @pl.when(i > 0)
    def _():
        s_ref[...] = s_ref[...] + su
        ss_ref[...] = ss_ref[...] + ssq


def _p3_body(out_ref, sc2_ref, sh2_ref, w2_ref, b2_ref, z_ref, s_ref, ss_ref):
    i = pl.program_id(0)
    t = out_ref[...] * sc2_ref[...] + sh2_ref[...]
    y = jnp.where(t >= 0.0, t, 0.1 * t)
    z = _mm(y, w2_ref[...]) + b2_ref[...]
    z_ref[...] = z
    su = jnp.sum(z, axis=0, keepdims=True)
    ssq = jnp.sum(z * z, axis=0, keepdims=True)

    @pl.when(i == 0)
    def _():
        s_ref[...] = su
        ss_ref[...] = ssq

    @pl.when(i > 0)
    def _():
        s_ref[...] = s_ref[...] + su
        ss_ref[...] = ss_ref[...] + ssq


def _p4_body(z_ref, sc3_ref, sh3_ref, x_ref):
    x_ref[...] = jnp.maximum(z_ref[...] * sc3_ref[...] + sh3_ref[...], 0.0)


def _bn_coeffs(s, ss, n, g, b):
    m = s / n
    v = jnp.maximum(ss / n - m * m, 0.0)
    sc = g[None, :] / jnp.sqrt(v + EPS)
    sh = b[None, :] - m * sc
    return sc, sh


def kernel(voxels, coors, num_points_per_voxel, pts, W1, b1, g1, be1,
           kpoints, Wk, gk, bk, W2, b2, g2, be2):
    del pts
    f32 = jnp.float32
    npv = num_points_per_voxel.astype(f32).reshape(V, 1)
    b1r = b1[None, :]
    kpx = kpoints[:, 0][None, :]
    kpy = kpoints[:, 1][None, :]
    kpz = kpoints[:, 2][None, :]
    wkf = Wk.reshape(M * COUT, COUT)
    b2r = b2[None, :]

    n_t = V // TILE_V
    anchors, s1, ss1 = pl.pallas_call(
        _p1_body,
        grid=(n_t,),
        in_specs=[
            pl.BlockSpec((TILE_V, P, 7), lambda i: (i, 0, 0)),
            pl.BlockSpec((TILE_V, 4), lambda i: (i, 0)),
            pl.BlockSpec((TILE_V, 1), lambda i: (i, 0)),
            pl.BlockSpec((11, COUT), lambda i: (0, 0)),
            pl.BlockSpec((1, COUT), lambda i: (0, 0)),
        ],
        out_specs=[
            pl.BlockSpec((TILE_V, 3), lambda i: (i, 0)),
            pl.BlockSpec((1, COUT), lambda i: (0, 0)),
            pl.BlockSpec((1, COUT), lambda i: (0, 0)),
        ],
        out_shape=[
            jax.ShapeDtypeStruct((V, 3), f32),
            jax.ShapeDtypeStruct((1, COUT), f32),
            jax.ShapeDtypeStruct((1, COUT), f32),
        ],
    )(voxels, coors, npv, W1, b1r)

    sc1, sh1 = _bn_coeffs(s1, ss1, N1, g1, be1)

    out, s2, ss2 = pl.pallas_call(
        _p2_body,
        grid=(n_t,),
        in_specs=[
            pl.BlockSpec((TILE_V, P, 7), lambda i: (i, 0, 0)),
            pl.BlockSpec((TILE_V, 4), lambda i: (i, 0)),
            pl.BlockSpec((TILE_V, 1), lambda i: (i, 0)),
            pl.BlockSpec((11, COUT), lambda i: (0, 0)),
            pl.BlockSpec((1, COUT), lambda i: (0, 0)),
            pl.BlockSpec((1, COUT), lambda i: (0, 0)),
            pl.BlockSpec((1, COUT), lambda i: (0, 0)),
            pl.BlockSpec((1, M), lambda i: (0, 0)),
            pl.BlockSpec((1, M), lambda i: (0, 0)),
            pl.BlockSpec((1, M), lambda i: (0, 0)),
            pl.BlockSpec((M * COUT, COUT), lambda i: (0, 0)),
        ],
        out_specs=[
            pl.BlockSpec((TILE_V, COUT), lambda i: (i, 0)),
            pl.BlockSpec((1, COUT), lambda i: (0, 0)),
            pl.BlockSpec((1, COUT), lambda i: (0, 0)),
        ],
        out_shape=[
            jax.ShapeDtypeStruct((V, COUT), f32),
            jax.ShapeDtypeStruct((1, COUT), f32),
            jax.ShapeDtypeStruct((1, COUT), f32),
        ],
    )(voxels, coors, npv, W1, b1r, sc1, sh1, kpx, kpy, kpz, wkf)

    sc2, sh2 = _bn_coeffs(s2, ss2, V, gk, bk)

    n_r = V // TILE_R
    z, s3, ss3 = pl.pallas_call(
        _p3_body,
        grid=(n_r,),
        in_specs=[
            pl.BlockSpec((TILE_R, COUT), lambda i: (i, 0)),
            pl.BlockSpec((1, COUT), lambda i: (0, 0)),
            pl.BlockSpec((1, COUT), lambda i: (0, 0)),
            pl.BlockSpec((COUT, COUT), lambda i: (0, 0)),
            pl.BlockSpec((1, COUT), lambda i: (0, 0)),
        ],
        out_specs=[
            pl.BlockSpec((TILE_R, COUT), lambda i: (i, 0)),
            pl.BlockSpec((1, COUT), lambda i: (0, 0)),
            pl.BlockSpec((1, COUT), lambda i: (0, 0)),
        ],
        out_shape=[
            jax.ShapeDtypeStruct((V, COUT), f32),
            jax.ShapeDtypeStruct((1, COUT), f32),
            jax.ShapeDtypeStruct((1, COUT), f32),
        ],
    )(out, sc2, sh2, W2, b2r)

    sc3, sh3 = _bn_coeffs(s3, ss3, V, g2, be2)

    x = pl.pallas_call(
        _p4_body,
        grid=(n_r,),
        in_specs=[
            pl.BlockSpec((TILE_R, COUT), lambda i: (i, 0)),
            pl.BlockSpec((1, COUT), lambda i: (0, 0)),
            pl.BlockSpec((1, COUT), lambda i: (0, 0)),
        ],
        out_specs=pl.BlockSpec((TILE_R, COUT), lambda i: (i, 0)),
        out_shape=jax.ShapeDtypeStruct((V, COUT), f32),
    )(z, sc3, sh3)

    return (anchors, x)


# speedup vs baseline: 5.8828x; 5.8828x over previous
import jax
import jax.numpy as jnp
from jax.experimental import pallas as pl

_HI = jax.lax.Precision.HIGHEST


def _mm(a, b):
    return jnp.dot(a, b, precision=_HI, preferred_element_type=jnp.float32)


V = 20000
P = 32
COUT = 64
M = 15
SIGMA = 0.3
EPS = 1e-5
VPAD = 20480
TILE_V1 = 2048
TILE_V2 = 400
TILE_R = 2000
N1 = V * P
NF = 11
NS = 1 + NF * P


def _p1_body(xr_ref, yr_ref, zr_ref, f0_ref, f1_ref, f2_ref, f3_ref,
             co_ref, npv_ref, valid_ref, thr2_ref, anch_ref, act_ref, s_ref):
    i = pl.program_id(0)
    co = co_ref[...]
    ax = co[3:4, :] * 0.2 + (0.1 - 51.2)
    ay = co[2:3, :] * 0.2 + (0.1 - 51.2)
    az = co[1:2, :] * 8.0 + (4.0 - 5.0)
    anch_ref[...] = jnp.concatenate([ax, ay, az], axis=0)
    x0, y0, z0 = xr_ref[...], yr_ref[...], zr_ref[...]
    empty = (x0 == 0.0) & (y0 == 0.0) & (z0 == 0.0)
    x = jnp.where(empty, ax, x0)
    y = jnp.where(empty, ay, y0)
    z = jnp.where(empty, az, z0)
    adx = x - ax
    ady = y - ay
    adz = z - az
    r2 = adx * adx + ady * ady + adz * adz
    rmin2 = jnp.min(r2, axis=0, keepdims=True)
    act_ref[...] = (rmin2 < thr2_ref[...]).astype(jnp.float32)
    cx = jnp.mean(x, axis=0, keepdims=True)
    cy = jnp.mean(y, axis=0, keepdims=True)
    cdx = x - cx
    cdy = y - cy
    shp = (P, x.shape[1])
    xs = jnp.concatenate([
        jnp.ones((1, x.shape[1]), jnp.float32),
        f0_ref[...], f1_ref[...], f2_ref[...], f3_ref[...],
        adx, ady, cdx, cdy,
        jnp.broadcast_to(cx, shp), jnp.broadcast_to(cy, shp),
        jnp.broadcast_to(npv_ref[...], shp),
    ], axis=0)
    xs = xs * valid_ref[...]
    s = jax.lax.dot_general(xs, xs, (((1,), (1,)), ((), ())),
                            precision=_HI, preferred_element_type=jnp.float32)

    @pl.when(i == 0)
    def _():
        s_ref[...] = s

    @pl.when(i > 0)
    def _():
        s_ref[...] = s_ref[...] + s


def _build_features(vox, coors_f, npv):
    ax = coors_f[:, 3:4] * 0.2 + (0.1 - 51.2)
    ay = coors_f[:, 2:3] * 0.2 + (0.1 - 51.2)
    az = coors_f[:, 1:2] * 8.0 + (4.0 - 5.0)
    anchors = jnp.concatenate([ax, ay, az], axis=1)
    xyz_raw = vox[:, :, 0:3]
    empty = ((vox[:, :, 0:1] == 0.0) & (vox[:, :, 1:2] == 0.0)
             & (vox[:, :, 2:3] == 0.0))
    xyz = jnp.where(empty, anchors[:, None, :], xyz_raw)
    feats = vox[:, :, 3:7]
    adiff = xyz - anchors[:, None, :]
    cent = jnp.mean(xyz, axis=1, keepdims=True)
    cdiff = xyz - cent
    cexp = jnp.broadcast_to(cent, xyz.shape)
    npx = jnp.broadcast_to(npv[:, :, None], (xyz.shape[0], P, 1))
    x11 = jnp.concatenate(
        [feats, adiff[:, :, :2], cdiff[:, :, :2], cexp[:, :, :2], npx], axis=2)
    return xyz, x11, anchors


def _p2_body(flag_ref, vox_ref, coors_ref, npv_ref, w1_ref, b1_ref,
             sc1_ref, sh1_ref, kpx_ref, kpy_ref, kpz_ref, wkf_ref,
             out_ref, s_ref, ss_ref):
    i = pl.program_id(0)
    flag = flag_ref[0, 0, 0]

    @pl.when(i == 0)
    def _():
        s_ref[...] = jnp.zeros_like(s_ref)
        ss_ref[...] = jnp.zeros_like(ss_ref)

    @pl.when(flag > 0.5)
    def _():
        coors_f = coors_ref[...].astype(jnp.float32)
        xyz, x11_3, anchors = _build_features(vox_ref[...], coors_f,
                                              npv_ref[...])
        x11 = x11_3.reshape(TILE_V2 * P, NF)
        u = _mm(x11, w1_ref[...]) + b1_ref[...]
        h = jnp.maximum(u * sc1_ref[...] + sh1_ref[...], 0.0)
        rel = (xyz - anchors[:, None, :]).reshape(TILE_V2 * P, 3)
        dx = rel[:, 0:1] - kpx_ref[...]
        dy = rel[:, 1:2] - kpy_ref[...]
        dz = rel[:, 2:3] - kpz_ref[...]
        d = jnp.sqrt(dx * dx + dy * dy + dz * dz)
        w = jnp.maximum(1.0 - d * (1.0 / SIGMA), 0.0)
        h3 = h.reshape(TILE_V2, P, COUT)
        w3 = w.reshape(TILE_V2, P, M)
        aggs = [jnp.sum(w3[:, :, m:m + 1] * h3, axis=1) for m in range(M)]
        agg = jnp.concatenate(aggs, axis=1)
        out = _mm(agg, wkf_ref[...])
        out_ref[...] = out
        s_ref[...] = s_ref[...] + jnp.sum(out, axis=0, keepdims=True)
        ss_ref[...] = ss_ref[...] + jnp.sum(out * out, axis=0, keepdims=True)

    @pl.when(flag <= 0.5)
    def _():
        out_ref[...] = jnp.zeros_like(out_ref)


def _p3_body(out_ref, sc2_ref, sh2_ref, w2_ref, b2_ref, z_ref, s_ref, ss_ref):
    i = pl.program_id(0)
    t = out_ref[...] * sc2_ref[...] + sh2_ref[...]
    y = jnp.where(t >= 0.0, t, 0.1 * t)
    z = _mm(y, w2_ref[...]) + b2_ref[...]
    z_ref[...] = z
    su = jnp.sum(z, axis=0, keepdims=True)
    ssq = jnp.sum(z * z, axis=0, keepdims=True)

    @pl.when(i == 0)
    def _():
        s_ref[...] = su
        ss_ref[...] = ssq

    @pl.when(i > 0)
    def _():
        s_ref[...] = s_ref[...] + su
        ss_ref[...] = ss_ref[...] + ssq


def _p4_body(z_ref, sc3_ref, sh3_ref, x_ref):
    x_ref[...] = jnp.maximum(z_ref[...] * sc3_ref[...] + sh3_ref[...], 0.0)


def _bn_from_sums(s, ss, n, g, b):
    m = s / n
    v = jnp.maximum(ss / n - m * m, 0.0)
    sc = g[None, :] / jnp.sqrt(v + EPS)
    sh = b[None, :] - m * sc
    return sc, sh


def kernel(voxels, coors, num_points_per_voxel, pts, W1, b1, g1, be1,
           kpoints, Wk, gk, bk, W2, b2, g2, be2):
    del pts
    f32 = jnp.float32
    npv = num_points_per_voxel.astype(f32).reshape(V, 1)
    b1r = b1[None, :]
    kpx = kpoints[:, 0][None, :]
    kpy = kpoints[:, 1][None, :]
    kpz = kpoints[:, 2][None, :]
    wkf = Wk.reshape(M * COUT, COUT)
    b2r = b2[None, :]

    pad = ((0, 0), (0, VPAD - V))
    xr = jnp.pad(voxels[:, :, 0].T, pad)
    yr = jnp.pad(voxels[:, :, 1].T, pad)
    zr = jnp.pad(voxels[:, :, 2].T, pad)
    f0 = jnp.pad(voxels[:, :, 3].T, pad)
    f1 = jnp.pad(voxels[:, :, 4].T, pad)
    f2 = jnp.pad(voxels[:, :, 5].T, pad)
    f3 = jnp.pad(voxels[:, :, 6].T, pad)
    coT = jnp.pad(coors.T.astype(f32), pad)
    npvT = jnp.pad(npv.T, pad)
    valid = jnp.pad(jnp.ones((1, V), f32), pad)
    kpmax = jnp.sqrt(jnp.max(jnp.sum(kpoints * kpoints, axis=1)))
    thr2 = ((SIGMA + kpmax + 1e-4) ** 2).reshape(1, 1)

    n_t1 = VPAD // TILE_V1
    vspec = pl.BlockSpec((P, TILE_V1), lambda i: (0, i))
    rspec = pl.BlockSpec((1, TILE_V1), lambda i: (0, i))
    anchT, actT, smom = pl.pallas_call(
        _p1_body,
        grid=(n_t1,),
        in_specs=[vspec, vspec, vspec, vspec, vspec, vspec, vspec,
                  pl.BlockSpec((4, TILE_V1), lambda i: (0, i)),
                  rspec, rspec,
                  pl.BlockSpec((1, 1), lambda i: (0, 0))],
        out_specs=[
            pl.BlockSpec((3, TILE_V1), lambda i: (0, i)),
            rspec,
            pl.BlockSpec((NS, NS), lambda i: (0, 0)),
        ],
        out_shape=[
            jax.ShapeDtypeStruct((3, VPAD), f32),
            jax.ShapeDtypeStruct((1, VPAD), f32),
            jax.ShapeDtypeStruct((NS, NS), f32),
        ],
    )(xr, yr, zr, f0, f1, f2, f3, coT, npvT, valid, thr2)

    anchors = anchT[:, :V].T
    act = actT[0, :V]

    sx = smom[0, 1:].reshape(NF, P).sum(axis=1)
    sxx = jnp.einsum('apbp->ab', smom[1:, 1:].reshape(NF, P, NF, P))
    mx = sx / N1
    cov = sxx / N1 - jnp.outer(mx, mx)
    mean_u = mx @ W1 + b1
    var_u = jnp.maximum(jnp.sum(W1 * (cov @ W1), axis=0), 0.0)
    sc1 = (g1 / jnp.sqrt(var_u + EPS))[None, :]
    sh1 = (be1 - mean_u * (g1 / jnp.sqrt(var_u + EPS)))[None, :]

    order = jnp.argsort(-act)
    n_act = jnp.sum(act)
    voxels_s = voxels[order]
    coors_s = coors[order]
    npv_s = npv[order]
    n_t2 = V // TILE_V2
    flags = ((jnp.arange(n_t2, dtype=f32) * TILE_V2) < n_act).astype(f32)
    flags = flags.reshape(n_t2, 1, 1)

    out_s, s2, ss2 = pl.pallas_call(
        _p2_body,
        grid=(n_t2,),
        in_specs=[
            pl.BlockSpec((1, 1, 1), lambda i: (i, 0, 0)),
            pl.BlockSpec((TILE_V2, P, 7), lambda i: (i, 0, 0)),
            pl.BlockSpec((TILE_V2, 4), lambda i: (i, 0)),
            pl.BlockSpec((TILE_V2, 1), lambda i: (i, 0)),
            pl.BlockSpec((NF, COUT), lambda i: (0, 0)),
            pl.BlockSpec((1, COUT), lambda i: (0, 0)),
            pl.BlockSpec((1, COUT), lambda i: (0, 0)),
            pl.BlockSpec((1, COUT), lambda i: (0, 0)),
            pl.BlockSpec((1, M), lambda i: (0, 0)),
            pl.BlockSpec((1, M), lambda i: (0, 0)),
            pl.BlockSpec((1, M), lambda i: (0, 0)),
            pl.BlockSpec((M * COUT, COUT), lambda i: (0, 0)),
        ],
        out_specs=[
            pl.BlockSpec((TILE_V2, COUT), lambda i: (i, 0)),
            pl.BlockSpec((1, COUT), lambda i: (0, 0)),
            pl.BlockSpec((1, COUT), lambda i: (0, 0)),
        ],
        out_shape=[
            jax.ShapeDtypeStruct((V, COUT), f32),
            jax.ShapeDtypeStruct((1, COUT), f32),
            jax.ShapeDtypeStruct((1, COUT), f32),
        ],
    )(flags, voxels_s, coors_s, npv_s, W1, b1r, sc1, sh1, kpx, kpy, kpz, wkf)

    sc2, sh2 = _bn_from_sums(s2, ss2, V, gk, bk)

    n_r = V // TILE_R
    z, s3, ss3 = pl.pallas_call(
        _p3_body,
        grid=(n_r,),
        in_specs=[
            pl.BlockSpec((TILE_R, COUT), lambda i: (i, 0)),
            pl.BlockSpec((1, COUT), lambda i: (0, 0)),
            pl.BlockSpec((1, COUT), lambda i: (0, 0)),
            pl.BlockSpec((COUT, COUT), lambda i: (0, 0)),
            pl.BlockSpec((1, COUT), lambda i: (0, 0)),
        ],
        out_specs=[
            pl.BlockSpec((TILE_R, COUT), lambda i: (i, 0)),
            pl.BlockSpec((1, COUT), lambda i: (0, 0)),
            pl.BlockSpec((1, COUT), lambda i: (0, 0)),
        ],
        out_shape=[
            jax.ShapeDtypeStruct((V, COUT), f32),
            jax.ShapeDtypeStruct((1, COUT), f32),
            jax.ShapeDtypeStruct((1, COUT), f32),
        ],
    )(out_s, sc2, sh2, W2, b2r)

    sc3, sh3 = _bn_from_sums(s3, ss3, V, g2, be2)

    x_s = pl.pallas_call(
        _p4_body,
        grid=(n_r,),
        in_specs=[
            pl.BlockSpec((TILE_R, COUT), lambda i: (i, 0)),
            pl.BlockSpec((1, COUT), lambda i: (0, 0)),
            pl.BlockSpec((1, COUT), lambda i: (0, 0)),
        ],
        out_specs=pl.BlockSpec((TILE_R, COUT), lambda i: (i, 0)),
        out_shape=jax.ShapeDtypeStruct((V, COUT), f32),
    )(z, sc3, sh3)

    inv = jnp.argsort(order)
    x = x_s[inv]
    return (anchors, x)


# speedup vs baseline: 6.6010x; 1.1221x over previous
import jax
import jax.numpy as jnp
from jax.experimental import pallas as pl
from jax.experimental.pallas import tpu as pltpu

_HI = jax.lax.Precision.HIGHEST


def _mm(a, b):
    return jnp.dot(a, b, precision=_HI, preferred_element_type=jnp.float32)


V = 20000
P = 32
COUT = 64
M = 15
SIGMA = 0.3
EPS = 1e-5
VPAD = 20480
TILE_V1 = 2048
TILE_V2 = 400
TILE_R = 2000
N1 = V * P
NF = 11
NS = 1 + NF * P


def _p1_body(xr_ref, yr_ref, zr_ref, f0_ref, f1_ref, f2_ref, f3_ref,
             co_ref, npv_ref, valid_ref, thr2_ref, anch_ref, act_ref, s_ref):
    i = pl.program_id(0)
    co = co_ref[...]
    cxy = jnp.float32(0.1) + jnp.float32(-51.2)
    cz = jnp.float32(4.0) + jnp.float32(-5.0)
    ax = co[3:4, :] * 0.2 + cxy
    ay = co[2:3, :] * 0.2 + cxy
    az = co[1:2, :] * 8.0 + cz
    anch_ref[...] = jnp.concatenate([ax, ay, az], axis=0)
    x0, y0, z0 = xr_ref[...], yr_ref[...], zr_ref[...]
    empty = (x0 == 0.0) & (y0 == 0.0) & (z0 == 0.0)
    x = jnp.where(empty, ax, x0)
    y = jnp.where(empty, ay, y0)
    z = jnp.where(empty, az, z0)
    adx = x - ax
    ady = y - ay
    adz = z - az
    r2 = adx * adx + ady * ady + adz * adz
    rmin2 = jnp.min(r2, axis=0, keepdims=True)
    act_ref[...] = (rmin2 < thr2_ref[...]).astype(jnp.float32)
    cx = jnp.mean(x, axis=0, keepdims=True)
    cy = jnp.mean(y, axis=0, keepdims=True)
    cdx = x - cx
    cdy = y - cy
    shp = (P, x.shape[1])
    xs = jnp.concatenate([
        jnp.ones((1, x.shape[1]), jnp.float32),
        f0_ref[...], f1_ref[...], f2_ref[...], f3_ref[...],
        adx, ady, cdx, cdy,
        jnp.broadcast_to(cx, shp), jnp.broadcast_to(cy, shp),
        jnp.broadcast_to(npv_ref[...], shp),
    ], axis=0)
    xs = xs * valid_ref[...]
    s = jax.lax.dot_general(xs, xs, (((1,), (1,)), ((), ())),
                            precision=_HI, preferred_element_type=jnp.float32)

    @pl.when(i == 0)
    def _():
        s_ref[...] = s

    @pl.when(i > 0)
    def _():
        s_ref[...] = s_ref[...] + s


def _build_features(vox, coors_f, npv):
    cxy = jnp.float32(0.1) + jnp.float32(-51.2)
    cz = jnp.float32(4.0) + jnp.float32(-5.0)
    ax = coors_f[:, 3:4] * 0.2 + cxy
    ay = coors_f[:, 2:3] * 0.2 + cxy
    az = coors_f[:, 1:2] * 8.0 + cz
    anchors = jnp.concatenate([ax, ay, az], axis=1)
    xyz_raw = vox[:, :, 0:3]
    empty = ((vox[:, :, 0:1] == 0.0) & (vox[:, :, 1:2] == 0.0)
             & (vox[:, :, 2:3] == 0.0))
    xyz = jnp.where(empty, anchors[:, None, :], xyz_raw)
    feats = vox[:, :, 3:7]
    adiff = xyz - anchors[:, None, :]
    cent = jnp.mean(xyz, axis=1, keepdims=True)
    cdiff = xyz - cent
    cexp = jnp.broadcast_to(cent, xyz.shape)
    npx = jnp.broadcast_to(npv[:, :, None], (xyz.shape[0], P, 1))
    x11 = jnp.concatenate(
        [feats, adiff[:, :, :2], cdiff[:, :, :2], cexp[:, :, :2], npx], axis=2)
    return xyz, x11, anchors


def _p2_body(flags_ref, vox_ref, coors_ref, npv_ref, w1_ref, b1_ref,
             sc1_ref, sh1_ref, kpx_ref, kpy_ref, kpz_ref, wkf_ref,
             out_ref, s_ref, ss_ref):
    i = pl.program_id(0)
    flag = flags_ref[i]

    @pl.when(i == 0)
    def _():
        s_ref[...] = jnp.zeros_like(s_ref)
        ss_ref[...] = jnp.zeros_like(ss_ref)

    @pl.when(flag > 0)
    def _():
        coors_f = coors_ref[...].astype(jnp.float32)
        xyz, x11_3, anchors = _build_features(vox_ref[...], coors_f,
                                              npv_ref[...])
        x11 = x11_3.reshape(TILE_V2 * P, NF)
        u = _mm(x11, w1_ref[...]) + b1_ref[...]
        h = jnp.maximum(u * sc1_ref[...] + sh1_ref[...], 0.0)
        rel = (xyz - anchors[:, None, :]).reshape(TILE_V2 * P, 3)
        dx = rel[:, 0:1] - kpx_ref[...]
        dy = rel[:, 1:2] - kpy_ref[...]
        dz = rel[:, 2:3] - kpz_ref[...]
        d = jnp.sqrt(dx * dx + dy * dy + dz * dz)
        w = jnp.maximum(1.0 - d * (1.0 / SIGMA), 0.0)
        h3 = h.reshape(TILE_V2, P, COUT)
        w3 = w.reshape(TILE_V2, P, M)
        aggs = [jnp.sum(w3[:, :, m:m + 1] * h3, axis=1) for m in range(M)]
        agg = jnp.concatenate(aggs, axis=1)
        out = _mm(agg, wkf_ref[...])
        out_ref[...] = out
        s_ref[...] = s_ref[...] + jnp.sum(out, axis=0, keepdims=True)
        ss_ref[...] = ss_ref[...] + jnp.sum(out * out, axis=0, keepdims=True)

    @pl.when(flag <= 0)
    def _():
        out_ref[...] = jnp.zeros_like(out_ref)


def _p3_body(out_ref, sc2_ref, sh2_ref, w2_ref, b2_ref, z_ref, s_ref, ss_ref):
    i = pl.program_id(0)
    t = out_ref[...] * sc2_ref[...] + sh2_ref[...]
    y = jnp.where(t >= 0.0, t, 0.1 * t)
    z = _mm(y, w2_ref[...]) + b2_ref[...]
    z_ref[...] = z
    su = jnp.sum(z, axis=0, keepdims=True)
    ssq = jnp.sum(z * z, axis=0, keepdims=True)

    @pl.when(i == 0)
    def _():
        s_ref[...] = su
        ss_ref[...] = ssq

    @pl.when(i > 0)
    def _():
        s_ref[...] = s_ref[...] + su
        ss_ref[...] = ss_ref[...] + ssq


def _p4_body(z_ref, sc3_ref, sh3_ref, x_ref):
    x_ref[...] = jnp.maximum(z_ref[...] * sc3_ref[...] + sh3_ref[...], 0.0)


def _bn_from_sums(s, ss, n, g, b):
    m = s / n
    v = jnp.maximum(ss / n - m * m, 0.0)
    sc = g[None, :] / jnp.sqrt(v + EPS)
    sh = b[None, :] - m * sc
    return sc, sh


def kernel(voxels, coors, num_points_per_voxel, pts, W1, b1, g1, be1,
           kpoints, Wk, gk, bk, W2, b2, g2, be2):
    del pts
    f32 = jnp.float32
    npv = num_points_per_voxel.astype(f32).reshape(V, 1)
    b1r = b1[None, :]
    kpx = kpoints[:, 0][None, :]
    kpy = kpoints[:, 1][None, :]
    kpz = kpoints[:, 2][None, :]
    wkf = Wk.reshape(M * COUT, COUT)
    b2r = b2[None, :]

    pad = ((0, 0), (0, VPAD - V))
    xr = jnp.pad(voxels[:, :, 0].T, pad)
    yr = jnp.pad(voxels[:, :, 1].T, pad)
    zr = jnp.pad(voxels[:, :, 2].T, pad)
    f0 = jnp.pad(voxels[:, :, 3].T, pad)
    f1 = jnp.pad(voxels[:, :, 4].T, pad)
    f2 = jnp.pad(voxels[:, :, 5].T, pad)
    f3 = jnp.pad(voxels[:, :, 6].T, pad)
    coT = jnp.pad(coors.T.astype(f32), pad)
    npvT = jnp.pad(npv.T, pad)
    valid = jnp.pad(jnp.ones((1, V), f32), pad)
    kpmax = jnp.sqrt(jnp.max(jnp.sum(kpoints * kpoints, axis=1)))
    thr2 = ((SIGMA + kpmax + 1e-4) ** 2).reshape(1, 1)

    n_t1 = VPAD // TILE_V1
    vspec = pl.BlockSpec((P, TILE_V1), lambda i: (0, i))
    rspec = pl.BlockSpec((1, TILE_V1), lambda i: (0, i))
    anchT, actT, smom = pl.pallas_call(
        _p1_body,
        grid=(n_t1,),
        in_specs=[vspec, vspec, vspec, vspec, vspec, vspec, vspec,
                  pl.BlockSpec((4, TILE_V1), lambda i: (0, i)),
                  rspec, rspec,
                  pl.BlockSpec((1, 1), lambda i: (0, 0))],
        out_specs=[
            pl.BlockSpec((3, TILE_V1), lambda i: (0, i)),
            rspec,
            pl.BlockSpec((NS, NS), lambda i: (0, 0)),
        ],
        out_shape=[
            jax.ShapeDtypeStruct((3, VPAD), f32),
            jax.ShapeDtypeStruct((1, VPAD), f32),
            jax.ShapeDtypeStruct((NS, NS), f32),
        ],
    )(xr, yr, zr, f0, f1, f2, f3, coT, npvT, valid, thr2)

    anchors = anchT[:, :V].T
    act = actT[0, :V]

    sx = smom[0, 1:].reshape(NF, P).sum(axis=1)
    sxx = jnp.einsum('apbp->ab', smom[1:, 1:].reshape(NF, P, NF, P))
    mx = sx / N1
    cov = sxx / N1 - jnp.outer(mx, mx)
    mean_u = mx @ W1 + b1
    var_u = jnp.maximum(jnp.sum(W1 * (cov @ W1), axis=0), 0.0)
    sc1 = (g1 / jnp.sqrt(var_u + EPS))[None, :]
    sh1 = (be1 - mean_u * (g1 / jnp.sqrt(var_u + EPS)))[None, :]

    order = jnp.argsort(-act)
    csum = jnp.cumsum(act)
    n_act = csum[V - 1]
    iota_v = jnp.arange(V, dtype=f32)
    inv = jnp.where(act > 0.5, csum - 1.0, n_act + iota_v - csum).astype(jnp.int32)
    voxels_s = voxels[order]
    coors_s = coors[order]
    npv_s = npv[order]
    n_t2 = V // TILE_V2
    flags = ((jnp.arange(n_t2, dtype=f32) * TILE_V2) < n_act).astype(jnp.int32)

    def _vix(i, fl):
        return (jnp.where(fl[i] > 0, i, 0), 0, 0)

    def _rix(i, fl):
        return (jnp.where(fl[i] > 0, i, 0), 0)

    def _zero2(i, fl):
        return (0, 0)

    grid2 = pltpu.PrefetchScalarGridSpec(
        num_scalar_prefetch=1,
        grid=(n_t2,),
        in_specs=[
            pl.BlockSpec((TILE_V2, P, 7), _vix),
            pl.BlockSpec((TILE_V2, 4), _rix),
            pl.BlockSpec((TILE_V2, 1), _rix),
            pl.BlockSpec((NF, COUT), _zero2),
            pl.BlockSpec((1, COUT), _zero2),
            pl.BlockSpec((1, COUT), _zero2),
            pl.BlockSpec((1, COUT), _zero2),
            pl.BlockSpec((1, M), _zero2),
            pl.BlockSpec((1, M), _zero2),
            pl.BlockSpec((1, M), _zero2),
            pl.BlockSpec((M * COUT, COUT), _zero2),
        ],
        out_specs=[
            pl.BlockSpec((TILE_V2, COUT), lambda i, fl: (i, 0)),
            pl.BlockSpec((1, COUT), _zero2),
            pl.BlockSpec((1, COUT), _zero2),
        ],
    )
    out_s, s2, ss2 = pl.pallas_call(
        _p2_body,
        grid_spec=grid2,
        out_shape=[
            jax.ShapeDtypeStruct((V, COUT), f32),
            jax.ShapeDtypeStruct((1, COUT), f32),
            jax.ShapeDtypeStruct((1, COUT), f32),
        ],
    )(flags, voxels_s, coors_s, npv_s, W1, b1r, sc1, sh1, kpx, kpy, kpz, wkf)

    sc2, sh2 = _bn_from_sums(s2, ss2, V, gk, bk)

    n_r = V // TILE_R
    z, s3, ss3 = pl.pallas_call(
        _p3_body,
        grid=(n_r,),
        in_specs=[
            pl.BlockSpec((TILE_R, COUT), lambda i: (i, 0)),
            pl.BlockSpec((1, COUT), lambda i: (0, 0)),
            pl.BlockSpec((1, COUT), lambda i: (0, 0)),
            pl.BlockSpec((COUT, COUT), lambda i: (0, 0)),
            pl.BlockSpec((1, COUT), lambda i: (0, 0)),
        ],
        out_specs=[
            pl.BlockSpec((TILE_R, COUT), lambda i: (i, 0)),
            pl.BlockSpec((1, COUT), lambda i: (0, 0)),
            pl.BlockSpec((1, COUT), lambda i: (0, 0)),
        ],
        out_shape=[
            jax.ShapeDtypeStruct((V, COUT), f32),
            jax.ShapeDtypeStruct((1, COUT), f32),
            jax.ShapeDtypeStruct((1, COUT), f32),
        ],
    )(out_s, sc2, sh2, W2, b2r)

    sc3, sh3 = _bn_from_sums(s3, ss3, V, g2, be2)

    x_s = pl.pallas_call(
        _p4_body,
        grid=(n_r,),
        in_specs=[
            pl.BlockSpec((TILE_R, COUT), lambda i: (i, 0)),
            pl.BlockSpec((1, COUT), lambda i: (0, 0)),
            pl.BlockSpec((1, COUT), lambda i: (0, 0)),
        ],
        out_specs=pl.BlockSpec((TILE_R, COUT), lambda i: (i, 0)),
        out_shape=jax.ShapeDtypeStruct((V, COUT), f32),
    )(z, sc3, sh3)

    x = x_s[inv]
    return (anchors, x)


# speedup vs baseline: 7.5620x; 1.1456x over previous
import jax
import jax.numpy as jnp
from jax.experimental import pallas as pl
from jax.experimental.pallas import tpu as pltpu

_HI = jax.lax.Precision.HIGHEST


def _mm(a, b):
    return jnp.dot(a, b, precision=_HI, preferred_element_type=jnp.float32)


V = 20000
P = 32
COUT = 64
M = 15
SIGMA = 0.3
EPS = 1e-5
VPAD = 20480
TILE_V1 = 2048
TILE_V2 = 200
TILE_R = 2000
N1 = V * P
NF = 11
NS = 1 + NF * P


def _p1_body(xr_ref, yr_ref, zr_ref, f0_ref, f1_ref, f2_ref, f3_ref,
             co_ref, npv_ref, valid_ref, thr2_ref, anch_ref, act_ref, s_ref):
    i = pl.program_id(0)
    co = co_ref[...]
    cxy = jnp.float32(0.1) + jnp.float32(-51.2)
    cz = jnp.float32(4.0) + jnp.float32(-5.0)
    ax = co[3:4, :] * 0.2 + cxy
    ay = co[2:3, :] * 0.2 + cxy
    az = co[1:2, :] * 8.0 + cz
    anch_ref[...] = jnp.concatenate([ax, ay, az], axis=0)
    x0, y0, z0 = xr_ref[...], yr_ref[...], zr_ref[...]
    empty = (x0 == 0.0) & (y0 == 0.0) & (z0 == 0.0)
    x = jnp.where(empty, ax, x0)
    y = jnp.where(empty, ay, y0)
    z = jnp.where(empty, az, z0)
    adx = x - ax
    ady = y - ay
    adz = z - az
    r2 = adx * adx + ady * ady + adz * adz
    rmin2 = jnp.min(r2, axis=0, keepdims=True)
    act_ref[...] = (rmin2 < thr2_ref[...]).astype(jnp.float32)
    cx = jnp.mean(x, axis=0, keepdims=True)
    cy = jnp.mean(y, axis=0, keepdims=True)
    cdx = x - cx
    cdy = y - cy
    shp = (P, x.shape[1])
    xs = jnp.concatenate([
        jnp.ones((1, x.shape[1]), jnp.float32),
        f0_ref[...], f1_ref[...], f2_ref[...], f3_ref[...],
        adx, ady, cdx, cdy,
        jnp.broadcast_to(cx, shp), jnp.broadcast_to(cy, shp),
        jnp.broadcast_to(npv_ref[...], shp),
    ], axis=0)
    xs = xs * valid_ref[...]
    s = jax.lax.dot_general(xs, xs, (((1,), (1,)), ((), ())),
                            precision=_HI, preferred_element_type=jnp.float32)

    @pl.when(i == 0)
    def _():
        s_ref[...] = s

    @pl.when(i > 0)
    def _():
        s_ref[...] = s_ref[...] + s


def _build_features(vox, coors_f, npv):
    cxy = jnp.float32(0.1) + jnp.float32(-51.2)
    cz = jnp.float32(4.0) + jnp.float32(-5.0)
    ax = coors_f[:, 3:4] * 0.2 + cxy
    ay = coors_f[:, 2:3] * 0.2 + cxy
    az = coors_f[:, 1:2] * 8.0 + cz
    anchors = jnp.concatenate([ax, ay, az], axis=1)
    xyz_raw = vox[:, :, 0:3]
    empty = ((vox[:, :, 0:1] == 0.0) & (vox[:, :, 1:2] == 0.0)
             & (vox[:, :, 2:3] == 0.0))
    xyz = jnp.where(empty, anchors[:, None, :], xyz_raw)
    feats = vox[:, :, 3:7]
    adiff = xyz - anchors[:, None, :]
    cent = jnp.mean(xyz, axis=1, keepdims=True)
    cdiff = xyz - cent
    cexp = jnp.broadcast_to(cent, xyz.shape)
    npx = jnp.broadcast_to(npv[:, :, None], (xyz.shape[0], P, 1))
    x11 = jnp.concatenate(
        [feats, adiff[:, :, :2], cdiff[:, :, :2], cexp[:, :, :2], npx], axis=2)
    return xyz, x11, anchors


def _p2_body(flags_ref, vox_ref, coors_ref, npv_ref, w1_ref, b1_ref,
             sc1_ref, sh1_ref, kpx_ref, kpy_ref, kpz_ref, wkf_ref,
             out_ref, s_ref, ss_ref):
    i = pl.program_id(0)
    flag = flags_ref[i]

    @pl.when(i == 0)
    def _():
        s_ref[...] = jnp.zeros_like(s_ref)
        ss_ref[...] = jnp.zeros_like(ss_ref)

    @pl.when(flag > 0)
    def _():
        coors_f = coors_ref[...].astype(jnp.float32)
        xyz, x11_3, anchors = _build_features(vox_ref[...], coors_f,
                                              npv_ref[...])
        x11 = x11_3.reshape(TILE_V2 * P, NF)
        u = _mm(x11, w1_ref[...]) + b1_ref[...]
        h = jnp.maximum(u * sc1_ref[...] + sh1_ref[...], 0.0)
        rel = (xyz - anchors[:, None, :]).reshape(TILE_V2 * P, 3)
        dx = rel[:, 0:1] - kpx_ref[...]
        dy = rel[:, 1:2] - kpy_ref[...]
        dz = rel[:, 2:3] - kpz_ref[...]
        d = jnp.sqrt(dx * dx + dy * dy + dz * dz)
        w = jnp.maximum(1.0 - d * (1.0 / SIGMA), 0.0)
        h3 = h.reshape(TILE_V2, P, COUT)
        w3 = w.reshape(TILE_V2, P, M)
        aggs = [jnp.sum(w3[:, :, m:m + 1] * h3, axis=1) for m in range(M)]
        agg = jnp.concatenate(aggs, axis=1)
        out = _mm(agg, wkf_ref[...])
        out_ref[...] = out
        s_ref[...] = s_ref[...] + jnp.sum(out, axis=0, keepdims=True)
        ss_ref[...] = ss_ref[...] + jnp.sum(out * out, axis=0, keepdims=True)

    @pl.when(flag <= 0)
    def _():
        out_ref[...] = jnp.zeros_like(out_ref)


def _p3_body(out_ref, sc2_ref, sh2_ref, w2_ref, b2_ref, z_ref, s_ref, ss_ref):
    i = pl.program_id(0)
    t = out_ref[...] * sc2_ref[...] + sh2_ref[...]
    y = jnp.where(t >= 0.0, t, 0.1 * t)
    z = _mm(y, w2_ref[...]) + b2_ref[...]
    z_ref[...] = z
    su = jnp.sum(z, axis=0, keepdims=True)
    ssq = jnp.sum(z * z, axis=0, keepdims=True)

    @pl.when(i == 0)
    def _():
        s_ref[...] = su
        ss_ref[...] = ssq

    @pl.when(i > 0)
    def _():
        s_ref[...] = s_ref[...] + su
        ss_ref[...] = ss_ref[...] + ssq


def _p4_body(z_ref, sc3_ref, sh3_ref, x_ref):
    x_ref[...] = jnp.maximum(z_ref[...] * sc3_ref[...] + sh3_ref[...], 0.0)


def _bn_from_sums(s, ss, n, g, b):
    m = s / n
    v = jnp.maximum(ss / n - m * m, 0.0)
    sc = g[None, :] / jnp.sqrt(v + EPS)
    sh = b[None, :] - m * sc
    return sc, sh


def kernel(voxels, coors, num_points_per_voxel, pts, W1, b1, g1, be1,
           kpoints, Wk, gk, bk, W2, b2, g2, be2):
    del pts
    f32 = jnp.float32
    npv = num_points_per_voxel.astype(f32).reshape(V, 1)
    b1r = b1[None, :]
    kpx = kpoints[:, 0][None, :]
    kpy = kpoints[:, 1][None, :]
    kpz = kpoints[:, 2][None, :]
    wkf = Wk.reshape(M * COUT, COUT)
    b2r = b2[None, :]

    pad = ((0, 0), (0, VPAD - V))
    xr = jnp.pad(voxels[:, :, 0].T, pad)
    yr = jnp.pad(voxels[:, :, 1].T, pad)
    zr = jnp.pad(voxels[:, :, 2].T, pad)
    f0 = jnp.pad(voxels[:, :, 3].T, pad)
    f1 = jnp.pad(voxels[:, :, 4].T, pad)
    f2 = jnp.pad(voxels[:, :, 5].T, pad)
    f3 = jnp.pad(voxels[:, :, 6].T, pad)
    coT = jnp.pad(coors.T.astype(f32), pad)
    npvT = jnp.pad(npv.T, pad)
    valid = jnp.pad(jnp.ones((1, V), f32), pad)
    kpmax = jnp.sqrt(jnp.max(jnp.sum(kpoints * kpoints, axis=1)))
    thr2 = ((SIGMA + kpmax + 1e-4) ** 2).reshape(1, 1)

    n_t1 = VPAD // TILE_V1
    vspec = pl.BlockSpec((P, TILE_V1), lambda i: (0, i))
    rspec = pl.BlockSpec((1, TILE_V1), lambda i: (0, i))
    anchT, actT, smom = pl.pallas_call(
        _p1_body,
        grid=(n_t1,),
        in_specs=[vspec, vspec, vspec, vspec, vspec, vspec, vspec,
                  pl.BlockSpec((4, TILE_V1), lambda i: (0, i)),
                  rspec, rspec,
                  pl.BlockSpec((1, 1), lambda i: (0, 0))],
        out_specs=[
            pl.BlockSpec((3, TILE_V1), lambda i: (0, i)),
            rspec,
            pl.BlockSpec((NS, NS), lambda i: (0, 0)),
        ],
        out_shape=[
            jax.ShapeDtypeStruct((3, VPAD), f32),
            jax.ShapeDtypeStruct((1, VPAD), f32),
            jax.ShapeDtypeStruct((NS, NS), f32),
        ],
    )(xr, yr, zr, f0, f1, f2, f3, coT, npvT, valid, thr2)

    anchors = anchT[:, :V].T
    act = actT[0, :V]

    sx = smom[0, 1:].reshape(NF, P).sum(axis=1)
    sxx = jnp.einsum('apbp->ab', smom[1:, 1:].reshape(NF, P, NF, P))
    mx = sx / N1
    cov = sxx / N1 - jnp.outer(mx, mx)
    mean_u = mx @ W1 + b1
    var_u = jnp.maximum(jnp.sum(W1 * (cov @ W1), axis=0), 0.0)
    sc1 = (g1 / jnp.sqrt(var_u + EPS))[None, :]
    sh1 = (be1 - mean_u * (g1 / jnp.sqrt(var_u + EPS)))[None, :]

    order = jnp.argsort(-act)
    csum = jnp.cumsum(act)
    n_act = csum[V - 1]
    iota_v = jnp.arange(V, dtype=f32)
    inv = jnp.where(act > 0.5, csum - 1.0, n_act + iota_v - csum).astype(jnp.int32)
    voxels_s = voxels[order]
    coors_s = coors[order]
    npv_s = npv[order]
    n_t2 = V // TILE_V2
    flags = ((jnp.arange(n_t2, dtype=f32) * TILE_V2) < n_act).astype(jnp.int32)

    def _vix(i, fl):
        return (jnp.where(fl[i] > 0, i, 0), 0, 0)

    def _rix(i, fl):
        return (jnp.where(fl[i] > 0, i, 0), 0)

    def _zero2(i, fl):
        return (0, 0)

    grid2 = pltpu.PrefetchScalarGridSpec(
        num_scalar_prefetch=1,
        grid=(n_t2,),
        in_specs=[
            pl.BlockSpec((TILE_V2, P, 7), _vix),
            pl.BlockSpec((TILE_V2, 4), _rix),
            pl.BlockSpec((TILE_V2, 1), _rix),
            pl.BlockSpec((NF, COUT), _zero2),
            pl.BlockSpec((1, COUT), _zero2),
            pl.BlockSpec((1, COUT), _zero2),
            pl.BlockSpec((1, COUT), _zero2),
            pl.BlockSpec((1, M), _zero2),
            pl.BlockSpec((1, M), _zero2),
            pl.BlockSpec((1, M), _zero2),
            pl.BlockSpec((M * COUT, COUT), _zero2),
        ],
        out_specs=[
            pl.BlockSpec((TILE_V2, COUT), lambda i, fl: (i, 0)),
            pl.BlockSpec((1, COUT), _zero2),
            pl.BlockSpec((1, COUT), _zero2),
        ],
    )
    out_s, s2, ss2 = pl.pallas_call(
        _p2_body,
        grid_spec=grid2,
        out_shape=[
            jax.ShapeDtypeStruct((V, COUT), f32),
            jax.ShapeDtypeStruct((1, COUT), f32),
            jax.ShapeDtypeStruct((1, COUT), f32),
        ],
    )(flags, voxels_s, coors_s, npv_s, W1, b1r, sc1, sh1, kpx, kpy, kpz, wkf)

    sc2, sh2 = _bn_from_sums(s2, ss2, V, gk, bk)

    n_r = V // TILE_R
    z, s3, ss3 = pl.pallas_call(
        _p3_body,
        grid=(n_r,),
        in_specs=[
            pl.BlockSpec((TILE_R, COUT), lambda i: (i, 0)),
            pl.BlockSpec((1, COUT), lambda i: (0, 0)),
            pl.BlockSpec((1, COUT), lambda i: (0, 0)),
            pl.BlockSpec((COUT, COUT), lambda i: (0, 0)),
            pl.BlockSpec((1, COUT), lambda i: (0, 0)),
        ],
        out_specs=[
            pl.BlockSpec((TILE_R, COUT), lambda i: (i, 0)),
            pl.BlockSpec((1, COUT), lambda i: (0, 0)),
            pl.BlockSpec((1, COUT), lambda i: (0, 0)),
        ],
        out_shape=[
            jax.ShapeDtypeStruct((V, COUT), f32),
            jax.ShapeDtypeStruct((1, COUT), f32),
            jax.ShapeDtypeStruct((1, COUT), f32),
        ],
    )(out_s, sc2, sh2, W2, b2r)

    sc3, sh3 = _bn_from_sums(s3, ss3, V, g2, be2)

    x_s = pl.pallas_call(
        _p4_body,
        grid=(n_r,),
        in_specs=[
            pl.BlockSpec((TILE_R, COUT), lambda i: (i, 0)),
            pl.BlockSpec((1, COUT), lambda i: (0, 0)),
            pl.BlockSpec((1, COUT), lambda i: (0, 0)),
        ],
        out_specs=pl.BlockSpec((TILE_R, COUT), lambda i: (i, 0)),
        out_shape=jax.ShapeDtypeStruct((V, COUT), f32),
    )(z, sc3, sh3)

    x = x_s[inv]
    return (anchors, x)
